# baseline (speedup 1.0000x reference)
import functools

import jax
import jax.numpy as jnp
from jax import lax
from jax.experimental import pallas as pl
from jax.experimental.pallas import tpu as pltpu
from jax.experimental.pallas import tpu_sc as plsc

_BATCH = 4096
_HIST = 200
_HIDDEN = 64
_B = _BATCH * _HIST
_CHUNK = 128


def _make_gather():
    info = plsc.get_sparse_core_info()
    nw = info.num_cores * info.num_subcores
    b_per_w = _B // nw
    n_chunks = b_per_w // _CHUNK
    mesh = plsc.VectorSubcoreMesh(core_axis_name="c", subcore_axis_name="s")

    @functools.partial(
        pl.kernel,
        mesh=mesh,
        out_type=jax.ShapeDtypeStruct((_B, _HIDDEN), jnp.float32),
        scratch_types=[
            pltpu.VMEM((n_chunks, _CHUNK), jnp.int32),
            pltpu.VMEM((_CHUNK, _HIDDEN), jnp.float32),
            pltpu.SemaphoreType.DMA,
        ],
        compiler_params=pltpu.CompilerParams(use_tc_tiling_on_sc=False),
    )
    def gather_kernel(idx_hbm, table_hbm, out_hbm, idx_v, rows_v, sem):
        wid = lax.axis_index("s") * info.num_cores + lax.axis_index("c")
        base = wid * b_per_w
        pltpu.sync_copy(idx_hbm.at[pl.ds(wid * n_chunks, n_chunks)], idx_v)

        def body(j, carry):
            pltpu.async_copy(table_hbm.at[idx_v.at[j]], rows_v, sem).wait()
            pltpu.sync_copy(rows_v, out_hbm.at[pl.ds(base + j * _CHUNK, _CHUNK)])
            return carry

        lax.fori_loop(0, n_chunks, body, 0)

    return gather_kernel


_gather = _make_gather()


def kernel(item_ids, table):
    idx = item_ids.reshape(-1).astype(jnp.int32).reshape(-1, _CHUNK)
    out = _gather(idx, table)
    return out.reshape(_BATCH, _HIST, _HIDDEN)


# speedup vs baseline: 1.1190x; 1.1190x over previous
import functools

import jax
import jax.numpy as jnp
from jax import lax
from jax.experimental import pallas as pl
from jax.experimental.pallas import tpu as pltpu
from jax.experimental.pallas import tpu_sc as plsc

_BATCH = 4096
_HIST = 200
_HIDDEN = 64
_B = _BATCH * _HIST
_IDXW = 128
_KPC = 4
_CHUNK = _IDXW * _KPC


def _make_gather():
    info = plsc.get_sparse_core_info()
    nw = info.num_cores * info.num_subcores
    b_per_w = _B // nw
    n_chunks = b_per_w // _CHUNK
    n_idx_rows = b_per_w // _IDXW
    mesh = plsc.VectorSubcoreMesh(core_axis_name="c", subcore_axis_name="s")

    @functools.partial(
        pl.kernel,
        mesh=mesh,
        out_type=jax.ShapeDtypeStruct((_B, _HIDDEN), jnp.float32),
        scratch_types=[
            pltpu.VMEM((n_idx_rows, _IDXW), jnp.int32),
            pltpu.VMEM((_CHUNK, _HIDDEN), jnp.float32),
            pltpu.VMEM((_CHUNK, _HIDDEN), jnp.float32),
            pltpu.SemaphoreType.DMA,
            pltpu.SemaphoreType.DMA,
            pltpu.SemaphoreType.DMA,
            pltpu.SemaphoreType.DMA,
        ],
        compiler_params=pltpu.CompilerParams(use_tc_tiling_on_sc=False),
    )
    def gather_kernel(idx_hbm, table_hbm, out_hbm, idx_v, rows0, rows1,
                      si0, si1, so0, so1):
        wid = lax.axis_index("s") * info.num_cores + lax.axis_index("c")
        base = wid * b_per_w
        pltpu.sync_copy(idx_hbm.at[pl.ds(wid * n_idx_rows, n_idx_rows)], idx_v)

        def fire(j, rows, sem):
            for k in range(_KPC):
                pltpu.async_copy(
                    table_hbm.at[idx_v.at[j * _KPC + k]],
                    rows.at[pl.ds(k * _IDXW, _IDXW)],
                    sem,
                )

        def drain(j, rows, sem):
            for k in range(_KPC):
                pltpu.make_async_copy(
                    table_hbm.at[idx_v.at[j * _KPC + k]],
                    rows.at[pl.ds(k * _IDXW, _IDXW)],
                    sem,
                ).wait()

        def write(j, rows, sem):
            return pltpu.async_copy(
                rows, out_hbm.at[pl.ds(base + j * _CHUNK, _CHUNK)], sem)

        def wait_write(j, rows, sem):
            pltpu.make_async_copy(
                rows, out_hbm.at[pl.ds(base + j * _CHUNK, _CHUNK)], sem).wait()

        bufs = ((rows0, si0, so0), (rows1, si1, so1))

        def body(jj, carry):
            for b in range(2):
                rows, si, so = bufs[b]
                o_rows, o_si, o_so = bufs[1 - b]
                j = 2 * jj + b

                @pl.when(j >= 2)
                def _():
                    wait_write(j - 2, rows, so)

                fire(j, rows, si)

                @pl.when(j >= 1)
                def _():
                    drain(j - 1, o_rows, o_si)
                    write(j - 1, o_rows, o_so)

            return carry

        lax.fori_loop(0, n_chunks // 2, body, 0)

        last = n_chunks - 1
        rows, si, so = bufs[last % 2]
        o_rows, o_si, o_so = bufs[1 - last % 2]
        drain(last, rows, si)
        write(last, rows, so)
        wait_write(last - 1, o_rows, o_so)
        wait_write(last, rows, so)

    return gather_kernel


_gather = _make_gather()


def kernel(item_ids, table):
    idx = item_ids.reshape(-1).astype(jnp.int32).reshape(-1, _IDXW)
    out = _gather(idx, table)
    return out.reshape(_BATCH, _HIST, _HIDDEN)
